# baseline (speedup 1.0000x reference)
import functools

import jax
import jax.numpy as jnp
from jax import lax
from jax.experimental import pallas as pl
from jax.experimental.pallas import tpu as pltpu
from jax.experimental.pallas import tpu_sc as plsc

B = 16384
D = 64
CH = 256


def _sc_gather_body(user_hbm, item_hbm, ut2_hbm, it2_hbm, uout_hbm, iout_hbm,
                    idx_u, idx_i, pidx_u, pidx_i, r128_u, r128_i, r64_u,
                    r64_i, sem_u, sem_i, nc, bpw):
    wid = lax.axis_index("s") * nc + lax.axis_index("c")
    base = wid * bpw
    pltpu.sync_copy(user_hbm.at[pl.ds(base, bpw)], idx_u)
    pltpu.sync_copy(item_hbm.at[pl.ds(base, bpw)], idx_i)

    def prep(g, _):
        gb = g * 16
        vu = idx_u[pl.ds(gb, 16)]
        vi = idx_i[pl.ds(gb, 16)]
        pidx_u[pl.ds(gb, 16)] = lax.shift_right_logical(vu, 1)
        pidx_i[pl.ds(gb, 16)] = lax.shift_right_logical(vi, 1)
        return 0

    lax.fori_loop(0, bpw // 16, prep, 0)

    for c in range(bpw // CH):
        cb = c * CH
        cu = pltpu.async_copy(
            ut2_hbm.at[pidx_u.at[pl.ds(cb, CH)]], r128_u, sem_u)
        ci = pltpu.async_copy(
            it2_hbm.at[pidx_i.at[pl.ds(cb, CH)]], r128_i, sem_i)
        cu.wait()
        ci.wait()

        def sel(g, _, cb=cb):
            gb = g * 16
            vu = idx_u[pl.ds(cb + gb, 16)]
            vi = idx_i[pl.ds(cb + gb, 16)]
            for jj in range(16):
                row = gb + jj
                hu = (vu[jj] & 1) * 64
                hi = (vi[jj] & 1) * 64
                for q in range(4):
                    r64_u[pl.ds(row * 64 + q * 16, 16)] = (
                        r128_u[row, pl.ds(hu + q * 16, 16)])
                    r64_i[pl.ds(row * 64 + q * 16, 16)] = (
                        r128_i[row, pl.ds(hi + q * 16, 16)])
            return 0

        lax.fori_loop(0, CH // 16, sel, 0)
        pltpu.sync_copy(r64_u, uout_hbm.at[pl.ds((base + cb) * 64, CH * 64)])
        pltpu.sync_copy(r64_i, iout_hbm.at[pl.ds((base + cb) * 64, CH * 64)])


@jax.jit
def _sc_gather(user, item, ut2, it2):
    info = plsc.get_sparse_core_info()
    nc, ns = info.num_cores, info.num_subcores
    nw = nc * ns
    bpw = B // nw
    mesh = plsc.VectorSubcoreMesh(core_axis_name="c", subcore_axis_name="s")
    body = functools.partial(_sc_gather_body, nc=nc, bpw=bpw)
    k = pl.kernel(
        body,
        out_type=[
            jax.ShapeDtypeStruct((B * D,), jnp.float32),
            jax.ShapeDtypeStruct((B * D,), jnp.float32),
        ],
        mesh=mesh,
        compiler_params=pltpu.CompilerParams(use_tc_tiling_on_sc=True),
        scratch_types=[
            pltpu.VMEM((bpw,), jnp.int32),
            pltpu.VMEM((bpw,), jnp.int32),
            pltpu.VMEM((bpw,), jnp.int32),
            pltpu.VMEM((bpw,), jnp.int32),
            pltpu.VMEM((CH, 128), jnp.float32),
            pltpu.VMEM((CH, 128), jnp.float32),
            pltpu.VMEM((CH * 64,), jnp.float32),
            pltpu.VMEM((CH * 64,), jnp.float32),
            pltpu.SemaphoreType.DMA,
            pltpu.SemaphoreType.DMA,
        ],
    )
    return k(user, item, ut2, it2)


def _mlp_body(u_ref, i_ref, w1a_ref, w1b_ref, b1_ref, w2_ref, b2_ref,
              w3_ref, b3_ref, out_ref):
    u = u_ref[...]
    i = i_ref[...]
    h = u @ w1a_ref[...] + i @ w1b_ref[...] + b1_ref[...]
    h = jnp.maximum(h, 0.0)
    h = jnp.maximum(h @ w2_ref[...] + b2_ref[...], 0.0)
    out_ref[...] = h @ w3_ref[...] + b3_ref[...]


@jax.jit
def _mlp(u, i, W1, b1, W2, b2, W3, b3):
    blk = 4096
    grid = B // blk
    w1a = W1[:D]
    w1b = W1[D:]
    full = lambda s: pl.BlockSpec(s, lambda j: (0, 0))
    out = pl.pallas_call(
        _mlp_body,
        grid=(grid,),
        in_specs=[
            pl.BlockSpec((blk, D), lambda j: (j, 0)),
            pl.BlockSpec((blk, D), lambda j: (j, 0)),
            full((D, 64)),
            full((D, 64)),
            full((1, 64)),
            full((64, 32)),
            full((1, 32)),
            full((32, 1)),
            full((1, 1)),
        ],
        out_specs=pl.BlockSpec((blk, 1), lambda j: (j, 0)),
        out_shape=jax.ShapeDtypeStruct((B, 1), jnp.float32),
    )(u, i, w1a, w1b, b1.reshape(1, 64), W2, b2.reshape(1, 32), W3,
      b3.reshape(1, 1))
    return out


def kernel(user, item, user_table, item_table, W1, b1, W2, b2, W3, b3):
    user = user.astype(jnp.int32)
    item = item.astype(jnp.int32)
    nu = user_table.shape[0] - 1
    ni = item_table.shape[0] - 1
    ut2 = user_table.reshape(-1)[: nu * D].reshape(nu // 2, 2 * D)
    it2 = item_table.reshape(-1)[: ni * D].reshape(ni // 2, 2 * D)
    uo, io = _sc_gather(user, item, ut2, it2)
    out = _mlp(uo.reshape(B, D), io.reshape(B, D), W1, b1, W2, b2, W3, b3)
    return jnp.squeeze(out, axis=-1)


# speedup vs baseline: 2.2738x; 2.2738x over previous
import functools

import jax
import jax.numpy as jnp
from jax import lax
from jax.experimental import pallas as pl
from jax.experimental.pallas import tpu as pltpu
from jax.experimental.pallas import tpu_sc as plsc

B = 16384
D = 64
CH = 256
CB = 12800
SPLIT_U = 512000
SPLIT_I = 51200


def _pack_body(top_ref, bot_ref, out_ref):
    t = jnp.transpose(top_ref[...])
    b = jnp.transpose(bot_ref[...])
    out_ref[...] = jnp.concatenate([t, b], axis=1)


def _pack(tT, split):
    grid = split // CB
    nblk = -(-tT.shape[1] // CB) - 1
    return pl.pallas_call(
        _pack_body,
        grid=(grid,),
        in_specs=[
            pl.BlockSpec((D, CB), lambda j: (0, j)),
            pl.BlockSpec((D, CB), lambda j, g=grid, n=nblk: (0, jnp.minimum(j + g, n))),
        ],
        out_specs=pl.BlockSpec((CB, 2 * D), lambda j: (j, 0)),
        out_shape=jax.ShapeDtypeStruct((split, 2 * D), jnp.float32),
    )(tT, tT)


def _sc_gather_body(user_hbm, item_hbm, utP_hbm, itP_hbm, uout_hbm, iout_hbm,
                    idx_u, idx_i, pidx_u, pidx_i, r128_u, r128_i, r64_u,
                    r64_i, sem_u, sem_i, nc, bpw):
    wid = lax.axis_index("s") * nc + lax.axis_index("c")
    base = wid * bpw
    pltpu.sync_copy(user_hbm.at[pl.ds(base, bpw)], idx_u)
    pltpu.sync_copy(item_hbm.at[pl.ds(base, bpw)], idx_i)

    def prep(g, _):
        gb = g * 16
        vu = idx_u[pl.ds(gb, 16)]
        vi = idx_i[pl.ds(gb, 16)]
        pidx_u[pl.ds(gb, 16)] = vu - jnp.where(vu >= SPLIT_U, SPLIT_U, 0)
        pidx_i[pl.ds(gb, 16)] = vi - jnp.where(vi >= SPLIT_I, SPLIT_I, 0)
        return 0

    lax.fori_loop(0, bpw // 16, prep, 0)

    for c in range(bpw // CH):
        cb = c * CH
        cu = pltpu.async_copy(
            utP_hbm.at[pidx_u.at[pl.ds(cb, CH)]], r128_u, sem_u)
        ci = pltpu.async_copy(
            itP_hbm.at[pidx_i.at[pl.ds(cb, CH)]], r128_i, sem_i)
        cu.wait()
        ci.wait()

        def sel(g, _, cb=cb):
            gb = g * 16
            vu = idx_u[pl.ds(cb + gb, 16)]
            vi = idx_i[pl.ds(cb + gb, 16)]
            for jj in range(16):
                row = gb + jj
                hu = jnp.where(vu[jj] >= SPLIT_U, D, 0)
                hi = jnp.where(vi[jj] >= SPLIT_I, D, 0)
                for q in range(4):
                    r64_u[pl.ds(row * 64 + q * 16, 16)] = (
                        r128_u[row, pl.ds(hu + q * 16, 16)])
                    r64_i[pl.ds(row * 64 + q * 16, 16)] = (
                        r128_i[row, pl.ds(hi + q * 16, 16)])
            return 0

        lax.fori_loop(0, CH // 16, sel, 0)
        pltpu.sync_copy(r64_u, uout_hbm.at[pl.ds((base + cb) * 64, CH * 64)])
        pltpu.sync_copy(r64_i, iout_hbm.at[pl.ds((base + cb) * 64, CH * 64)])


@jax.jit
def _sc_gather(user, item, utP, itP):
    info = plsc.get_sparse_core_info()
    nc, ns = info.num_cores, info.num_subcores
    nw = nc * ns
    bpw = B // nw
    mesh = plsc.VectorSubcoreMesh(core_axis_name="c", subcore_axis_name="s")
    body = functools.partial(_sc_gather_body, nc=nc, bpw=bpw)
    k = pl.kernel(
        body,
        out_type=[
            jax.ShapeDtypeStruct((B * D,), jnp.float32),
            jax.ShapeDtypeStruct((B * D,), jnp.float32),
        ],
        mesh=mesh,
        compiler_params=pltpu.CompilerParams(use_tc_tiling_on_sc=True),
        scratch_types=[
            pltpu.VMEM((bpw,), jnp.int32),
            pltpu.VMEM((bpw,), jnp.int32),
            pltpu.VMEM((bpw,), jnp.int32),
            pltpu.VMEM((bpw,), jnp.int32),
            pltpu.VMEM((CH, 128), jnp.float32),
            pltpu.VMEM((CH, 128), jnp.float32),
            pltpu.VMEM((CH * 64,), jnp.float32),
            pltpu.VMEM((CH * 64,), jnp.float32),
            pltpu.SemaphoreType.DMA,
            pltpu.SemaphoreType.DMA,
        ],
    )
    return k(user, item, utP, itP)


def _mlp_body(u_ref, i_ref, w1a_ref, w1b_ref, b1_ref, w2_ref, b2_ref,
              w3_ref, b3_ref, out_ref):
    u = u_ref[...]
    i = i_ref[...]
    h = u @ w1a_ref[...] + i @ w1b_ref[...] + b1_ref[...]
    h = jnp.maximum(h, 0.0)
    h = jnp.maximum(h @ w2_ref[...] + b2_ref[...], 0.0)
    out_ref[...] = h @ w3_ref[...] + b3_ref[...]


@jax.jit
def _mlp(u, i, W1, b1, W2, b2, W3, b3):
    blk = 4096
    grid = B // blk
    w1a = W1[:D]
    w1b = W1[D:]
    full = lambda s: pl.BlockSpec(s, lambda j: (0, 0))
    out = pl.pallas_call(
        _mlp_body,
        grid=(grid,),
        in_specs=[
            pl.BlockSpec((blk, D), lambda j: (j, 0)),
            pl.BlockSpec((blk, D), lambda j: (j, 0)),
            full((D, 64)),
            full((D, 64)),
            full((1, 64)),
            full((64, 32)),
            full((1, 32)),
            full((32, 1)),
            full((1, 1)),
        ],
        out_specs=pl.BlockSpec((blk, 1), lambda j: (j, 0)),
        out_shape=jax.ShapeDtypeStruct((B, 1), jnp.float32),
    )(u, i, w1a, w1b, b1.reshape(1, 64), W2, b2.reshape(1, 32), W3,
      b3.reshape(1, 1))
    return out


def kernel(user, item, user_table, item_table, W1, b1, W2, b2, W3, b3):
    user = user.astype(jnp.int32)
    item = item.astype(jnp.int32)
    utP = _pack(user_table.T, SPLIT_U)
    itP = _pack(item_table.T, SPLIT_I)
    uo, io = _sc_gather(user, item, utP, itP)
    out = _mlp(uo.reshape(B, D), io.reshape(B, D), W1, b1, W2, b2, W3, b3)
    return jnp.squeeze(out, axis=-1)


# speedup vs baseline: 2.5781x; 1.1338x over previous
import functools

import jax
import jax.numpy as jnp
from jax import lax
from jax.experimental import pallas as pl
from jax.experimental.pallas import tpu as pltpu
from jax.experimental.pallas import tpu_sc as plsc

B = 16384
D = 64
CH = 256
CB = 12800
SPLIT_U = 512000
SPLIT_I = 51200


def _pack_body(top_ref, bot_ref, elo_ref, ehi_ref, out_ref):
    cn = (((0,), (0,)), ((), ()))
    out_ref[...] = (lax.dot_general(top_ref[...], elo_ref[...], cn) +
                    lax.dot_general(bot_ref[...], ehi_ref[...], cn))


def _pack(tT, split, elo, ehi):
    grid = split // CB
    nblk = -(-tT.shape[1] // CB) - 1
    return pl.pallas_call(
        _pack_body,
        grid=(grid,),
        in_specs=[
            pl.BlockSpec((D, CB), lambda j: (0, j)),
            pl.BlockSpec((D, CB), lambda j, g=grid, n=nblk: (0, jnp.minimum(j + g, n))),
            pl.BlockSpec((D, 2 * D), lambda j: (0, 0)),
            pl.BlockSpec((D, 2 * D), lambda j: (0, 0)),
        ],
        out_specs=pl.BlockSpec((CB, 2 * D), lambda j: (j, 0)),
        out_shape=jax.ShapeDtypeStruct((split, 2 * D), jnp.float32),
    )(tT, tT, elo, ehi)


def _sc_gather_body(user_hbm, item_hbm, utP_hbm, itP_hbm, uout_hbm, iout_hbm,
                    idx_u, idx_i, pidx_u, pidx_i, r128_u, r128_i, r64_u,
                    r64_i, sem_u, sem_i, nc, bpw):
    wid = lax.axis_index("s") * nc + lax.axis_index("c")
    base = wid * bpw
    pltpu.sync_copy(user_hbm.at[pl.ds(base, bpw)], idx_u)
    pltpu.sync_copy(item_hbm.at[pl.ds(base, bpw)], idx_i)

    def prep(g, _):
        gb = g * 16
        vu = idx_u[pl.ds(gb, 16)]
        vi = idx_i[pl.ds(gb, 16)]
        pidx_u[pl.ds(gb, 16)] = vu - jnp.where(vu >= SPLIT_U, SPLIT_U, 0)
        pidx_i[pl.ds(gb, 16)] = vi - jnp.where(vi >= SPLIT_I, SPLIT_I, 0)
        return 0

    lax.fori_loop(0, bpw // 16, prep, 0)

    for c in range(bpw // CH):
        cb = c * CH
        cu = pltpu.async_copy(
            utP_hbm.at[pidx_u.at[pl.ds(cb, CH)]], r128_u, sem_u)
        ci = pltpu.async_copy(
            itP_hbm.at[pidx_i.at[pl.ds(cb, CH)]], r128_i, sem_i)
        cu.wait()
        ci.wait()

        def sel(g, _, cb=cb):
            gb = g * 16
            vu = idx_u[pl.ds(cb + gb, 16)]
            vi = idx_i[pl.ds(cb + gb, 16)]
            for jj in range(16):
                row = gb + jj
                hu = jnp.where(vu[jj] >= SPLIT_U, D, 0)
                hi = jnp.where(vi[jj] >= SPLIT_I, D, 0)
                for q in range(4):
                    r64_u[pl.ds(row * 64 + q * 16, 16)] = (
                        r128_u[row, pl.ds(hu + q * 16, 16)])
                    r64_i[pl.ds(row * 64 + q * 16, 16)] = (
                        r128_i[row, pl.ds(hi + q * 16, 16)])
            return 0

        lax.fori_loop(0, CH // 16, sel, 0)
        pltpu.sync_copy(r64_u, uout_hbm.at[pl.ds((base + cb) * 64, CH * 64)])
        pltpu.sync_copy(r64_i, iout_hbm.at[pl.ds((base + cb) * 64, CH * 64)])


@jax.jit
def _sc_gather(user, item, utP, itP):
    info = plsc.get_sparse_core_info()
    nc, ns = info.num_cores, info.num_subcores
    nw = nc * ns
    bpw = B // nw
    mesh = plsc.VectorSubcoreMesh(core_axis_name="c", subcore_axis_name="s")
    body = functools.partial(_sc_gather_body, nc=nc, bpw=bpw)
    k = pl.kernel(
        body,
        out_type=[
            jax.ShapeDtypeStruct((B * D,), jnp.float32),
            jax.ShapeDtypeStruct((B * D,), jnp.float32),
        ],
        mesh=mesh,
        compiler_params=pltpu.CompilerParams(use_tc_tiling_on_sc=True),
        scratch_types=[
            pltpu.VMEM((bpw,), jnp.int32),
            pltpu.VMEM((bpw,), jnp.int32),
            pltpu.VMEM((bpw,), jnp.int32),
            pltpu.VMEM((bpw,), jnp.int32),
            pltpu.VMEM((CH, 128), jnp.float32),
            pltpu.VMEM((CH, 128), jnp.float32),
            pltpu.VMEM((CH * 64,), jnp.float32),
            pltpu.VMEM((CH * 64,), jnp.float32),
            pltpu.SemaphoreType.DMA,
            pltpu.SemaphoreType.DMA,
        ],
    )
    return k(user, item, utP, itP)


def _mlp_body(u_ref, i_ref, w1a_ref, w1b_ref, b1_ref, w2_ref, b2_ref,
              w3_ref, b3_ref, out_ref):
    u = u_ref[...]
    i = i_ref[...]
    h = u @ w1a_ref[...] + i @ w1b_ref[...] + b1_ref[...]
    h = jnp.maximum(h, 0.0)
    h = jnp.maximum(h @ w2_ref[...] + b2_ref[...], 0.0)
    out_ref[...] = h @ w3_ref[...] + b3_ref[...]


@jax.jit
def _mlp(u, i, W1, b1, W2, b2, W3, b3):
    blk = 4096
    grid = B // blk
    w1a = W1[:D]
    w1b = W1[D:]
    full = lambda s: pl.BlockSpec(s, lambda j: (0, 0))
    out = pl.pallas_call(
        _mlp_body,
        grid=(grid,),
        in_specs=[
            pl.BlockSpec((blk, D), lambda j: (j, 0)),
            pl.BlockSpec((blk, D), lambda j: (j, 0)),
            full((D, 64)),
            full((D, 64)),
            full((1, 64)),
            full((64, 32)),
            full((1, 32)),
            full((32, 1)),
            full((1, 1)),
        ],
        out_specs=pl.BlockSpec((blk, 1), lambda j: (j, 0)),
        out_shape=jax.ShapeDtypeStruct((B, 1), jnp.float32),
    )(u, i, w1a, w1b, b1.reshape(1, 64), W2, b2.reshape(1, 32), W3,
      b3.reshape(1, 1))
    return out


def kernel(user, item, user_table, item_table, W1, b1, W2, b2, W3, b3):
    user = user.astype(jnp.int32)
    item = item.astype(jnp.int32)
    elo = jnp.eye(D, 2 * D, dtype=jnp.float32)
    ehi = jnp.eye(D, 2 * D, k=D, dtype=jnp.float32)
    utP = _pack(user_table.T, SPLIT_U, elo, ehi)
    itP = _pack(item_table.T, SPLIT_I, elo, ehi)
    uo, io = _sc_gather(user, item, utP, itP)
    out = _mlp(uo.reshape(B, D), io.reshape(B, D), W1, b1, W2, b2, W3, b3)
    return jnp.squeeze(out, axis=-1)
